# baseline (speedup 1.0000x reference)
import functools

import jax
import jax.numpy as jnp
from jax import lax
from jax.experimental import pallas as pl
from jax.experimental.pallas import tpu as pltpu
from jax.experimental.pallas import tpu_sc as plsc

B = 16384
D = 32
NC = 2
NS = 16
NW = NC * NS
BPW = B // NW

_sc_mesh = plsc.VectorSubcoreMesh(core_axis_name="c", subcore_axis_name="s")


@functools.partial(
    pl.kernel,
    mesh=_sc_mesh,
    compiler_params=pltpu.CompilerParams(use_tc_tiling_on_sc=False),
    out_type=[jax.ShapeDtypeStruct((B, D), jnp.float32) for _ in range(4)],
    scratch_types=[
        pltpu.VMEM((BPW,), jnp.int32),
        pltpu.VMEM((BPW,), jnp.int32),
        pltpu.VMEM((BPW, D), jnp.float32),
        pltpu.VMEM((BPW, D), jnp.float32),
        pltpu.VMEM((BPW, D), jnp.float32),
        pltpu.VMEM((BPW, D), jnp.float32),
        pltpu.SemaphoreType.DMA,
        pltpu.SemaphoreType.DMA,
        pltpu.SemaphoreType.DMA,
        pltpu.SemaphoreType.DMA,
    ],
)
def _sc_gather(user_hbm, item_hbm, ugt, igt, umt, imt,
               ug_o, ig_o, um_o, im_o,
               uidx, iidx, ug_v, ig_v, um_v, im_v, s0, s1, s2, s3):
    wid = lax.axis_index("s") * NC + lax.axis_index("c")
    base = wid * BPW
    pltpu.sync_copy(user_hbm.at[pl.ds(base, BPW)], uidx)
    pltpu.sync_copy(item_hbm.at[pl.ds(base, BPW)], iidx)
    c0 = pltpu.async_copy(ugt.at[uidx], ug_v, s0)
    c1 = pltpu.async_copy(igt.at[iidx], ig_v, s1)
    c2 = pltpu.async_copy(umt.at[uidx], um_v, s2)
    c3 = pltpu.async_copy(imt.at[iidx], im_v, s3)
    c0.wait()
    pltpu.sync_copy(ug_v, ug_o.at[pl.ds(base, BPW)])
    c1.wait()
    pltpu.sync_copy(ig_v, ig_o.at[pl.ds(base, BPW)])
    c2.wait()
    pltpu.sync_copy(um_v, um_o.at[pl.ds(base, BPW)])
    c3.wait()
    pltpu.sync_copy(im_v, im_o.at[pl.ds(base, BPW)])


_BLK = 2048


def _mlp_body(ug, ig, um, im, w1a, w1b, b1, w2, b2, wpg, wph, bp, out):
    gmf = ug[...] * ig[...]
    h = (jnp.dot(um[...], w1a[...], preferred_element_type=jnp.float32)
         + jnp.dot(im[...], w1b[...], preferred_element_type=jnp.float32)
         + b1[...])
    h = jnp.maximum(h, 0.0)
    h = jnp.dot(h, w2[...], preferred_element_type=jnp.float32) + b2[...]
    h = jnp.maximum(h, 0.0)
    y = (jnp.dot(gmf, wpg[...], preferred_element_type=jnp.float32)
         + jnp.dot(h, wph[...], preferred_element_type=jnp.float32)
         + bp[...])
    out[...] = y


_row_spec = pl.BlockSpec((_BLK, D), lambda i: (i, 0))


def _full(shape):
    return pl.BlockSpec(shape, lambda i: tuple(0 for _ in shape))


_mlp_call = pl.pallas_call(
    _mlp_body,
    grid=(B // _BLK,),
    in_specs=[
        _row_spec, _row_spec, _row_spec, _row_spec,
        _full((D, D)), _full((D, D)), _full((1, D)),
        _full((D, 16)), _full((1, 16)),
        _full((D, 1)), _full((16, 1)), _full((1, 1)),
    ],
    out_specs=pl.BlockSpec((_BLK, 1), lambda i: (i, 0)),
    out_shape=jax.ShapeDtypeStruct((B, 1), jnp.float32),
)


def kernel(user, item, user_gmf, item_gmf, user_mlp, item_mlp,
           W1, b1, W2, b2, Wp, bp):
    user = user.astype(jnp.int32)
    item = item.astype(jnp.int32)
    ug, ig, um, im = _sc_gather(user, item, user_gmf, item_gmf,
                                user_mlp, item_mlp)
    y = _mlp_call(ug, ig, um, im,
                  W1[:D], W1[D:], b1.reshape(1, D),
                  W2, b2.reshape(1, 16),
                  Wp[:D], Wp[D:], bp.reshape(1, 1))
    return y.reshape(B)
